# Initial kernel scaffold; baseline (speedup 1.0000x reference)
import jax
import jax.numpy as jnp
from jax.experimental import pallas as pl


def kernel(x, adj, W1, b1, W2, b2):
    raise NotImplementedError("write your pallas kernel here")


# speedup vs baseline: 28.0253x; 28.0253x over previous
import functools

import jax
import jax.numpy as jnp
from jax import lax
from jax.experimental import pallas as pl
from jax.experimental.pallas import tpu as pltpu
from jax.experimental.pallas import tpu_sc as plsc

N_NODES = 10000
D_IN = 128
D_HID = 32
N_EDGES = 320000

NC = 2
NS = 16
NW = NC * NS
NPAD = 10240
ROWS_PER_TILE = NPAD // NS
CHUNK = 80
CH_PER_W = N_EDGES // (NW * CHUNK)

_sc_mesh = plsc.VectorSubcoreMesh(core_axis_name="c", subcore_axis_name="s")
_sc_params = pltpu.CompilerParams(use_tc_tiling_on_sc=False)


@functools.partial(
    pl.kernel,
    out_type=jax.ShapeDtypeStruct((NC, NPAD), jnp.float32),
    mesh=_sc_mesh,
    compiler_params=_sc_params,
    scratch_types=[
        pltpu.VMEM((CHUNK,), jnp.float32),
        pltpu.VMEM((CH_PER_W, CHUNK), jnp.int32),
        pltpu.VMEM_SHARED((NPAD,), jnp.float32),
    ],
)
def _sc_degree(dst2d_hbm, ones_hbm, zeros1_hbm, out_hbm, ones_v, dst_all, deg_sh):
    c = lax.axis_index("c")
    s = lax.axis_index("s")
    w = c * NS + s
    pltpu.sync_copy(zeros1_hbm, deg_sh.at[pl.ds(s * ROWS_PER_TILE, ROWS_PER_TILE)])
    pltpu.sync_copy(ones_hbm, ones_v)
    pltpu.sync_copy(dst2d_hbm.at[w], dst_all)
    plsc.subcore_barrier()

    def body(it, carry):
        pltpu.sync_copy(ones_v, deg_sh.at[dst_all.at[it]], add=True)
        return carry

    lax.fori_loop(0, CH_PER_W, body, 0)
    plsc.subcore_barrier()
    sl = pl.ds(s * ROWS_PER_TILE, ROWS_PER_TILE)
    pltpu.sync_copy(deg_sh.at[sl], out_hbm.at[c, sl])


@functools.partial(
    pl.kernel,
    out_type=jax.ShapeDtypeStruct((NC, NPAD, D_HID), jnp.float32),
    mesh=_sc_mesh,
    compiler_params=_sc_params,
    scratch_types=[
        pltpu.VMEM((CH_PER_W, CHUNK), jnp.int32),
        pltpu.VMEM((CH_PER_W, CHUNK), jnp.int32),
        pltpu.VMEM((CHUNK, D_HID), jnp.float32),
        pltpu.VMEM_SHARED((NPAD, D_HID), jnp.float32),
    ],
)
def _sc_scatter(g_hbm, src2d_hbm, dst2d_hbm, zeros2_hbm, out_hbm,
                src_all, dst_all, rows_v, agg_sh):
    c = lax.axis_index("c")
    s = lax.axis_index("s")
    w = c * NS + s
    pltpu.sync_copy(zeros2_hbm, agg_sh.at[pl.ds(s * ROWS_PER_TILE, ROWS_PER_TILE)])
    pltpu.sync_copy(src2d_hbm.at[w], src_all)
    pltpu.sync_copy(dst2d_hbm.at[w], dst_all)
    plsc.subcore_barrier()

    def body(it, carry):
        pltpu.sync_copy(g_hbm.at[src_all.at[it]], rows_v)
        pltpu.sync_copy(rows_v, agg_sh.at[dst_all.at[it]], add=True)
        return carry

    lax.fori_loop(0, CH_PER_W, body, 0)
    plsc.subcore_barrier()
    sl = pl.ds(s * ROWS_PER_TILE, ROWS_PER_TILE)
    pltpu.sync_copy(agg_sh.at[sl], out_hbm.at[c, sl])


def _tc1_body(x_ref, w1_ref, degpt_ref, g1_ref, dis_ref):
    deg = degpt_ref[:, 0:1] + degpt_ref[:, 1:2] + 1.0
    dis = lax.rsqrt(deg)
    dis_ref[...] = dis
    h = jnp.dot(x_ref[...], w1_ref[...], preferred_element_type=jnp.float32)
    g1_ref[...] = dis * h


def _tc2_body(aggp_ref, g1_ref, dis_ref, b1_ref, g2_ref):
    dis = dis_ref[...]
    h1 = dis * (aggp_ref[0] + aggp_ref[1] + g1_ref[...]) + b1_ref[...]
    g2_ref[...] = dis * h1


def _tc3_body(aggp_ref, g2_ref, dis_ref, w2_ref, b2_ref, out_ref):
    pre = dis_ref[...] * (aggp_ref[0] + aggp_ref[1] + g2_ref[...])
    y = jnp.dot(pre, w2_ref[...], preferred_element_type=jnp.float32) + b2_ref[...]
    out_ref[...] = jnp.maximum(y, 0.0)


_tc1 = pl.pallas_call(
    _tc1_body,
    out_shape=(jax.ShapeDtypeStruct((NPAD, D_HID), jnp.float32),
               jax.ShapeDtypeStruct((NPAD, 1), jnp.float32)),
)
_tc2 = pl.pallas_call(
    _tc2_body,
    out_shape=jax.ShapeDtypeStruct((NPAD, D_HID), jnp.float32),
)
_tc3 = pl.pallas_call(
    _tc3_body,
    out_shape=jax.ShapeDtypeStruct((NPAD, D_IN), jnp.float32),
)


def kernel(x, adj, W1, b1, W2, b2):
    x = x.astype(jnp.float32)
    src = adj[0].astype(jnp.int32).reshape(NW, CH_PER_W, CHUNK)
    dst = adj[1].astype(jnp.int32).reshape(NW, CH_PER_W, CHUNK)
    xp = jnp.zeros((NPAD, D_IN), jnp.float32).at[:N_NODES].set(x)

    ones_ch = jnp.ones((CHUNK,), jnp.float32)
    zeros1 = jnp.zeros((ROWS_PER_TILE,), jnp.float32)
    zeros2 = jnp.zeros((ROWS_PER_TILE, D_HID), jnp.float32)

    degp = _sc_degree(dst, ones_ch, zeros1)
    degpt = degp.T

    g1, dis = _tc1(xp, W1, degpt)
    aggp1 = _sc_scatter(g1, src, dst, zeros2)
    g2 = _tc2(aggp1, g1, dis, b1[None, :])
    aggp2 = _sc_scatter(g2, src, dst, zeros2)
    out = _tc3(aggp2, g2, dis, W2, b2[None, :])
    return out[:N_NODES]


# speedup vs baseline: 30.8189x; 1.0997x over previous
import functools

import jax
import jax.numpy as jnp
from jax import lax
from jax.experimental import pallas as pl
from jax.experimental.pallas import tpu as pltpu
from jax.experimental.pallas import tpu_sc as plsc

N_NODES = 10000
D_IN = 128
D_HID = 32
N_EDGES = 320000

NC = 2
NS = 16
NW = NC * NS
NPAD = 10240
ROWS_PER_TILE = NPAD // NS
CHUNK = 128
CH_PER_W = 80
EPAD = NW * CH_PER_W * CHUNK
RING = 8
DIST = 4

_sc_mesh = plsc.VectorSubcoreMesh(core_axis_name="c", subcore_axis_name="s")
_sc_params = pltpu.CompilerParams(use_tc_tiling_on_sc=False)


@functools.partial(
    pl.kernel,
    out_type=jax.ShapeDtypeStruct((NC, NPAD), jnp.float32),
    mesh=_sc_mesh,
    compiler_params=_sc_params,
    scratch_types=[
        pltpu.VMEM((CHUNK,), jnp.float32),
        pltpu.VMEM((CH_PER_W, CHUNK), jnp.int32),
        pltpu.VMEM_SHARED((NPAD,), jnp.float32),
    ],
)
def _sc_degree(dst3d_hbm, ones_hbm, zeros1_hbm, out_hbm, ones_v, dst_all, deg_sh):
    c = lax.axis_index("c")
    s = lax.axis_index("s")
    w = c * NS + s
    pltpu.sync_copy(zeros1_hbm, deg_sh.at[pl.ds(s * ROWS_PER_TILE, ROWS_PER_TILE)])
    pltpu.sync_copy(ones_hbm, ones_v)
    pltpu.sync_copy(dst3d_hbm.at[w], dst_all)
    plsc.subcore_barrier()

    def body(it, carry):
        pltpu.sync_copy(ones_v, deg_sh.at[dst_all.at[it]], add=True)
        return carry

    lax.fori_loop(0, CH_PER_W, body, 0)
    plsc.subcore_barrier()
    sl = pl.ds(s * ROWS_PER_TILE, ROWS_PER_TILE)
    pltpu.sync_copy(deg_sh.at[sl], out_hbm.at[c, sl])


@functools.partial(
    pl.kernel,
    out_type=jax.ShapeDtypeStruct((NC, NPAD, D_HID), jnp.float32),
    mesh=_sc_mesh,
    compiler_params=_sc_params,
    scratch_types=[
        pltpu.VMEM((CH_PER_W, CHUNK), jnp.int32),
        pltpu.VMEM((CH_PER_W, CHUNK), jnp.int32),
        pltpu.VMEM((RING, CHUNK, D_HID), jnp.float32),
        pltpu.VMEM_SHARED((NPAD, D_HID), jnp.float32),
        pltpu.SemaphoreType.DMA((RING,)),
        pltpu.SemaphoreType.DMA((RING,)),
    ],
)
def _sc_scatter(g_hbm, src3d_hbm, dst3d_hbm, zeros2_hbm, out_hbm,
                src_all, dst_all, rows, agg_sh, gsem, ssem):
    c = lax.axis_index("c")
    s = lax.axis_index("s")
    w = c * NS + s
    pltpu.sync_copy(zeros2_hbm, agg_sh.at[pl.ds(s * ROWS_PER_TILE, ROWS_PER_TILE)])
    pltpu.sync_copy(src3d_hbm.at[w], src_all)
    pltpu.sync_copy(dst3d_hbm.at[w], dst_all)
    plsc.subcore_barrier()

    def gather_start(b, it):
        pltpu.async_copy(g_hbm.at[src_all.at[it]], rows.at[b], gsem.at[b])

    def gather_wait(b, it):
        pltpu.make_async_copy(g_hbm.at[src_all.at[it]], rows.at[b],
                              gsem.at[b]).wait()

    def scat_start(b, it):
        pltpu.async_copy(rows.at[b], agg_sh.at[dst_all.at[it]], ssem.at[b],
                         add=True)

    def scat_wait(b, it):
        pltpu.make_async_copy(rows.at[b], agg_sh.at[dst_all.at[it]],
                              ssem.at[b]).wait()

    for b in range(DIST):
        gather_start(b, b)

    def group(k, carry):
        for b in range(RING):
            it = k * RING + b
            gather_wait(b, it)
            scat_start(b, it)
            wb = (b + DIST) % RING
            if b >= DIST:
                scat_wait(wb, it - DIST)

                @pl.when(k < (CH_PER_W // RING) - 1)
                def _():
                    gather_start(wb, it + DIST)
            else:
                @pl.when(k > 0)
                def _():
                    scat_wait(wb, it - DIST)

                gather_start(wb, it + DIST)
        return carry

    lax.fori_loop(0, CH_PER_W // RING, group, 0)
    for it in range(CH_PER_W - DIST, CH_PER_W):
        scat_wait(it % RING, it)
    plsc.subcore_barrier()
    sl = pl.ds(s * ROWS_PER_TILE, ROWS_PER_TILE)
    pltpu.sync_copy(agg_sh.at[sl], out_hbm.at[c, sl])


def _tc1_body(x_ref, w1_ref, degpt_ref, g1_ref, dis_ref):
    deg = degpt_ref[:, 0:1] + degpt_ref[:, 1:2] + 1.0
    dis = lax.rsqrt(deg)
    dis_ref[...] = dis
    h = jnp.dot(x_ref[...], w1_ref[...], preferred_element_type=jnp.float32)
    g1_ref[...] = dis * h


def _tc2_body(aggp_ref, g1_ref, dis_ref, b1_ref, g2_ref):
    dis = dis_ref[...]
    h1 = dis * (aggp_ref[0] + aggp_ref[1] + g1_ref[...]) + b1_ref[...]
    g2_ref[...] = dis * h1


def _tc3_body(aggp_ref, g2_ref, dis_ref, w2_ref, b2_ref, out_ref):
    pre = dis_ref[...] * (aggp_ref[0] + aggp_ref[1] + g2_ref[...])
    y = jnp.dot(pre, w2_ref[...], preferred_element_type=jnp.float32) + b2_ref[...]
    out_ref[...] = jnp.maximum(y, 0.0)


_tc1 = pl.pallas_call(
    _tc1_body,
    out_shape=(jax.ShapeDtypeStruct((NPAD, D_HID), jnp.float32),
               jax.ShapeDtypeStruct((NPAD, 1), jnp.float32)),
)
_tc2 = pl.pallas_call(
    _tc2_body,
    out_shape=jax.ShapeDtypeStruct((NPAD, D_HID), jnp.float32),
)
_tc3 = pl.pallas_call(
    _tc3_body,
    out_shape=jax.ShapeDtypeStruct((NPAD, D_IN), jnp.float32),
)


def kernel(x, adj, W1, b1, W2, b2):
    x = x.astype(jnp.float32)
    pad = jnp.full((EPAD - N_EDGES,), NPAD - 1, jnp.int32)
    src = jnp.concatenate([adj[0].astype(jnp.int32), pad]).reshape(
        NW, CH_PER_W, CHUNK)
    dst = jnp.concatenate([adj[1].astype(jnp.int32), pad]).reshape(
        NW, CH_PER_W, CHUNK)
    xp = jnp.zeros((NPAD, D_IN), jnp.float32).at[:N_NODES].set(x)

    ones_ch = jnp.ones((CHUNK,), jnp.float32)
    zeros1 = jnp.zeros((ROWS_PER_TILE,), jnp.float32)
    zeros2 = jnp.zeros((ROWS_PER_TILE, D_HID), jnp.float32)

    degp = _sc_degree(dst, ones_ch, zeros1)
    degpt = degp.T

    g1, dis = _tc1(xp, W1, degpt)
    aggp1 = _sc_scatter(g1, src, dst, zeros2)
    g2 = _tc2(aggp1, g1, dis, b1[None, :])
    aggp2 = _sc_scatter(g2, src, dst, zeros2)
    out = _tc3(aggp2, g2, dis, W2, b2[None, :])
    return out[:N_NODES]


# speedup vs baseline: 53.3889x; 1.7323x over previous
import functools

import jax
import jax.numpy as jnp
from jax import lax
from jax.experimental import pallas as pl
from jax.experimental.pallas import tpu as pltpu
from jax.experimental.pallas import tpu_sc as plsc

N_NODES = 10000
D_IN = 128
D_HID = 32
N_EDGES = 320000

NC = 2
NS = 16
NW = NC * NS
NPAD = 10240
ROWS_PER_TILE = NPAD // NS
CHUNK = 128
CH_PER_W = 80
EPAD = NW * CH_PER_W * CHUNK
RING = 8
DIST = 4

_sc_mesh = plsc.VectorSubcoreMesh(core_axis_name="c", subcore_axis_name="s")
_sc_params = pltpu.CompilerParams(use_tc_tiling_on_sc=False)


@functools.partial(
    pl.kernel,
    out_type=jax.ShapeDtypeStruct((NC, NPAD), jnp.float32),
    mesh=_sc_mesh,
    compiler_params=_sc_params,
    scratch_types=[
        pltpu.VMEM((CHUNK,), jnp.float32),
        pltpu.VMEM((CH_PER_W, CHUNK), jnp.int32),
        pltpu.VMEM_SHARED((NPAD,), jnp.float32),
    ],
)
def _sc_degree(dst3d_hbm, ones_hbm, zeros1_hbm, out_hbm, ones_v, dst_all, deg_sh):
    c = lax.axis_index("c")
    s = lax.axis_index("s")
    w = c * NS + s
    pltpu.sync_copy(zeros1_hbm, deg_sh.at[pl.ds(s * ROWS_PER_TILE, ROWS_PER_TILE)])
    pltpu.sync_copy(ones_hbm, ones_v)
    pltpu.sync_copy(dst3d_hbm.at[w], dst_all)
    plsc.subcore_barrier()

    def body(it, carry):
        pltpu.sync_copy(ones_v, deg_sh.at[dst_all.at[it]], add=True)
        return carry

    lax.fori_loop(0, CH_PER_W, body, 0)
    plsc.subcore_barrier()
    sl = pl.ds(s * ROWS_PER_TILE, ROWS_PER_TILE)
    pltpu.sync_copy(deg_sh.at[sl], out_hbm.at[c, sl])


@functools.partial(
    pl.kernel,
    out_type=jax.ShapeDtypeStruct((NC, NPAD, D_HID), jnp.float32),
    mesh=_sc_mesh,
    compiler_params=_sc_params,
    scratch_types=[
        pltpu.VMEM((CH_PER_W, CHUNK), jnp.int32),
        pltpu.VMEM((CH_PER_W, CHUNK), jnp.int32),
        pltpu.VMEM((RING, CHUNK, D_HID), jnp.float32),
        pltpu.VMEM_SHARED((NPAD, D_HID), jnp.float32),
        pltpu.SemaphoreType.DMA((RING,)),
        pltpu.SemaphoreType.DMA((RING,)),
    ],
)
def _sc_scatter(g_hbm, src3d_hbm, dst3d_hbm, zeros2_hbm, out_hbm,
                src_all, dst_all, rows, agg_sh, gsem, ssem):
    c = lax.axis_index("c")
    s = lax.axis_index("s")
    w = c * NS + s
    pltpu.sync_copy(zeros2_hbm, agg_sh.at[pl.ds(s * ROWS_PER_TILE, ROWS_PER_TILE)])
    pltpu.sync_copy(src3d_hbm.at[w], src_all)
    pltpu.sync_copy(dst3d_hbm.at[w], dst_all)
    plsc.subcore_barrier()

    def gather_start(b, it):
        pltpu.async_copy(g_hbm.at[src_all.at[it]], rows.at[b], gsem.at[b])

    def gather_wait(b, it):
        pltpu.make_async_copy(g_hbm.at[src_all.at[it]], rows.at[b],
                              gsem.at[b]).wait()

    def scat_start(b, it):
        pltpu.async_copy(rows.at[b], agg_sh.at[dst_all.at[it]], ssem.at[b],
                         add=True)

    def scat_wait(b, it):
        pltpu.make_async_copy(rows.at[b], agg_sh.at[dst_all.at[it]],
                              ssem.at[b]).wait()

    for b in range(DIST):
        gather_start(b, b)

    def group(k, carry):
        for b in range(RING):
            it = k * RING + b
            gather_wait(b, it)
            scat_start(b, it)
            wb = (b + DIST) % RING
            if b >= DIST:
                scat_wait(wb, it - DIST)

                @pl.when(k < (CH_PER_W // RING) - 1)
                def _():
                    gather_start(wb, it + DIST)
            else:
                @pl.when(k > 0)
                def _():
                    scat_wait(wb, it - DIST)

                gather_start(wb, it + DIST)
        return carry

    lax.fori_loop(0, CH_PER_W // RING, group, 0)
    for it in range(CH_PER_W - DIST, CH_PER_W):
        scat_wait(it % RING, it)
    plsc.subcore_barrier()
    sl = pl.ds(s * ROWS_PER_TILE, ROWS_PER_TILE)
    pltpu.sync_copy(agg_sh.at[sl], out_hbm.at[c, sl])


def _tc1_body(x_ref, w1_ref, degpt_ref, g1_ref, dis_ref):
    deg = degpt_ref[:, 0:1] + degpt_ref[:, 1:2] + 1.0
    dis = lax.rsqrt(deg)
    dis_ref[...] = dis
    h = jnp.dot(x_ref[...], w1_ref[...], preferred_element_type=jnp.float32)
    g1_ref[...] = dis * h


def _tc2_body(aggp_ref, g1_ref, dis_ref, b1_ref, g2_ref):
    dis = dis_ref[...]
    h1 = dis * (aggp_ref[0] + aggp_ref[1] + g1_ref[...]) + b1_ref[...]
    g2_ref[...] = dis * h1


def _tc3_body(aggp_ref, g2_ref, dis_ref, w2_ref, b2_ref, out_ref):
    pre = dis_ref[...] * (aggp_ref[0] + aggp_ref[1] + g2_ref[...])
    y = jnp.dot(pre, w2_ref[...], preferred_element_type=jnp.float32) + b2_ref[...]
    out_ref[...] = jnp.maximum(y, 0.0)


_tc1 = pl.pallas_call(
    _tc1_body,
    out_shape=(jax.ShapeDtypeStruct((NPAD, D_HID), jnp.float32),
               jax.ShapeDtypeStruct((NPAD, 1), jnp.float32)),
)
_tc2 = pl.pallas_call(
    _tc2_body,
    out_shape=jax.ShapeDtypeStruct((NPAD, D_HID), jnp.float32),
)
_tc3 = pl.pallas_call(
    _tc3_body,
    out_shape=jax.ShapeDtypeStruct((NPAD, D_IN), jnp.float32),
)


def kernel(x, adj, W1, b1, W2, b2):
    x = x.astype(jnp.float32)
    pad = N_NODES + jnp.arange(EPAD - N_EDGES, dtype=jnp.int32) % (NPAD - N_NODES)
    src = jnp.concatenate([adj[0].astype(jnp.int32), pad]).reshape(
        NW, CH_PER_W, CHUNK)
    dst = jnp.concatenate([adj[1].astype(jnp.int32), pad]).reshape(
        NW, CH_PER_W, CHUNK)
    xp = jnp.zeros((NPAD, D_IN), jnp.float32).at[:N_NODES].set(x)

    ones_ch = jnp.ones((CHUNK,), jnp.float32)
    zeros1 = jnp.zeros((ROWS_PER_TILE,), jnp.float32)
    zeros2 = jnp.zeros((ROWS_PER_TILE, D_HID), jnp.float32)

    degp = _sc_degree(dst, ones_ch, zeros1)
    degpt = degp.T

    g1, dis = _tc1(xp, W1, degpt)
    aggp1 = _sc_scatter(g1, src, dst, zeros2)
    g2 = _tc2(aggp1, g1, dis, b1[None, :])
    aggp2 = _sc_scatter(g2, src, dst, zeros2)
    out = _tc3(aggp2, g2, dis, W2, b2[None, :])
    return out[:N_NODES]


# speedup vs baseline: 55.3017x; 1.0358x over previous
import functools

import jax
import jax.numpy as jnp
from jax import lax
from jax.experimental import pallas as pl
from jax.experimental.pallas import tpu as pltpu
from jax.experimental.pallas import tpu_sc as plsc

N_NODES = 10000
D_IN = 128
D_HID = 32
N_EDGES = 320000

NC = 2
NS = 16
NW = NC * NS
NPAD = 10240
ROWS_PER_TILE = NPAD // NS
CHUNK = 128
CH_PER_W = 80
EPAD = NW * CH_PER_W * CHUNK
RING = 8
DIST = 4
WROWS = NPAD * D_HID // 128

_sc_mesh = plsc.VectorSubcoreMesh(core_axis_name="c", subcore_axis_name="s")
_sc_params = pltpu.CompilerParams(use_tc_tiling_on_sc=False)


@functools.partial(
    pl.kernel,
    out_type=jax.ShapeDtypeStruct((NC, NPAD), jnp.float32),
    mesh=_sc_mesh,
    compiler_params=_sc_params,
    scratch_types=[
        pltpu.VMEM((CHUNK,), jnp.float32),
        pltpu.VMEM((CH_PER_W, CHUNK), jnp.int32),
        pltpu.VMEM_SHARED((NPAD,), jnp.float32),
    ],
)
def _sc_degree(dst2d_hbm, ones_hbm, zeros1_hbm, out_hbm, ones_v, dst_all, deg_sh):
    c = lax.axis_index("c")
    s = lax.axis_index("s")
    w = c * NS + s
    pltpu.sync_copy(zeros1_hbm, deg_sh.at[pl.ds(s * ROWS_PER_TILE, ROWS_PER_TILE)])
    pltpu.sync_copy(ones_hbm, ones_v)
    pltpu.sync_copy(dst2d_hbm.at[pl.ds(w * CH_PER_W, CH_PER_W)], dst_all)
    plsc.subcore_barrier()

    def body(it, carry):
        pltpu.sync_copy(ones_v, deg_sh.at[dst_all.at[it]], add=True)
        return carry

    lax.fori_loop(0, CH_PER_W, body, 0)
    plsc.subcore_barrier()
    sl = pl.ds(s * ROWS_PER_TILE, ROWS_PER_TILE)
    pltpu.sync_copy(deg_sh.at[sl], out_hbm.at[c, sl])


@functools.partial(
    pl.kernel,
    out_type=jax.ShapeDtypeStruct((NC, NPAD, D_HID), jnp.float32),
    mesh=_sc_mesh,
    compiler_params=_sc_params,
    scratch_types=[
        pltpu.VMEM((CH_PER_W, CHUNK), jnp.int32),
        pltpu.VMEM((CH_PER_W, CHUNK), jnp.int32),
        pltpu.VMEM((RING, CHUNK, D_HID), jnp.float32),
        pltpu.VMEM_SHARED((NPAD, D_HID), jnp.float32),
        pltpu.SemaphoreType.DMA((RING,)),
        pltpu.SemaphoreType.DMA((RING,)),
    ],
)
def _sc_scatter(g_hbm, src2d_hbm, dst2d_hbm, zeros2_hbm, out_hbm,
                src_all, dst_all, rows, agg_sh, gsem, ssem):
    c = lax.axis_index("c")
    s = lax.axis_index("s")
    w = c * NS + s
    pltpu.sync_copy(zeros2_hbm, agg_sh.at[pl.ds(s * ROWS_PER_TILE, ROWS_PER_TILE)])
    pltpu.sync_copy(src2d_hbm.at[pl.ds(w * CH_PER_W, CH_PER_W)], src_all)
    pltpu.sync_copy(dst2d_hbm.at[pl.ds(w * CH_PER_W, CH_PER_W)], dst_all)
    plsc.subcore_barrier()

    def gather_start(b, it):
        pltpu.async_copy(g_hbm.at[src_all.at[it]], rows.at[b], gsem.at[b])

    def gather_wait(b, it):
        pltpu.make_async_copy(g_hbm.at[src_all.at[it]], rows.at[b],
                              gsem.at[b]).wait()

    def scat_start(b, it):
        pltpu.async_copy(rows.at[b], agg_sh.at[dst_all.at[it]], ssem.at[b],
                         add=True)

    def scat_wait(b, it):
        pltpu.make_async_copy(rows.at[b], agg_sh.at[dst_all.at[it]],
                              ssem.at[b]).wait()

    for b in range(DIST):
        gather_start(b, b)

    def group(k, carry):
        for b in range(RING):
            it = k * RING + b
            gather_wait(b, it)
            scat_start(b, it)
            wb = (b + DIST) % RING
            if b >= DIST:
                scat_wait(wb, it - DIST)

                @pl.when(k < (CH_PER_W // RING) - 1)
                def _():
                    gather_start(wb, it + DIST)
            else:
                @pl.when(k > 0)
                def _():
                    scat_wait(wb, it - DIST)

                gather_start(wb, it + DIST)
        return carry

    lax.fori_loop(0, CH_PER_W // RING, group, 0)
    for it in range(CH_PER_W - DIST, CH_PER_W):
        scat_wait(it % RING, it)
    plsc.subcore_barrier()
    sl = pl.ds(s * ROWS_PER_TILE, ROWS_PER_TILE)
    pltpu.sync_copy(agg_sh.at[sl], out_hbm.at[c, sl])


def _tc1_body(x_ref, w1_ref, degpt_ref, g1_ref, dis_ref):
    deg = degpt_ref[:, 0:1] + degpt_ref[:, 1:2] + 1.0
    dis = lax.rsqrt(deg)
    dis_ref[...] = dis
    h = jnp.dot(x_ref[...], w1_ref[...], preferred_element_type=jnp.float32)
    g1_ref[0:N_NODES] = dis[:N_NODES] * h
    g1_ref[N_NODES:] = jnp.zeros((NPAD - N_NODES, D_HID), jnp.float32)


def _tc2_body(aggp_ref, g1_ref, dis_ref, b1_ref, g2_ref):
    dis = dis_ref[...]
    t = aggp_ref[0] + aggp_ref[1] + g1_ref[...]
    g2_ref[...] = dis * (dis * t + b1_ref[...])


def _tc3_body(aggp_ref, g2_ref, dis_ref, w2_ref, b2_ref, out_ref):
    t = aggp_ref[0] + aggp_ref[1] + g2_ref[...]
    pre = (dis_ref[...] * t)[:N_NODES]
    y = jnp.dot(pre, w2_ref[...], preferred_element_type=jnp.float32) + b2_ref[...]
    out_ref[...] = jnp.maximum(y, 0.0)


_tc1 = pl.pallas_call(
    _tc1_body,
    out_shape=(jax.ShapeDtypeStruct((NPAD, D_HID), jnp.float32),
               jax.ShapeDtypeStruct((NPAD, 1), jnp.float32)),
)
_tc2 = pl.pallas_call(
    _tc2_body,
    out_shape=jax.ShapeDtypeStruct((NPAD, D_HID), jnp.float32),
)
_tc3 = pl.pallas_call(
    _tc3_body,
    out_shape=jax.ShapeDtypeStruct((N_NODES, D_IN), jnp.float32),
)


def kernel(x, adj, W1, b1, W2, b2):
    x = x.astype(jnp.float32)
    pad = N_NODES + jnp.arange(EPAD - N_EDGES, dtype=jnp.int32) % (NPAD - N_NODES)
    src = jnp.concatenate([adj[0].astype(jnp.int32), pad]).reshape(
        NW * CH_PER_W, CHUNK)
    dst = jnp.concatenate([adj[1].astype(jnp.int32), pad]).reshape(
        NW * CH_PER_W, CHUNK)

    ones_ch = jnp.ones((CHUNK,), jnp.float32)
    zeros1 = jnp.zeros((ROWS_PER_TILE,), jnp.float32)
    zeros2 = jnp.zeros((ROWS_PER_TILE, D_HID), jnp.float32)

    degp = _sc_degree(dst, ones_ch, zeros1)
    degpt = degp.T

    g1, dis = _tc1(x, W1, degpt)
    aggp1 = _sc_scatter(g1, src, dst, zeros2)
    g2 = _tc2(aggp1, g1, dis, b1[None, :])
    aggp2 = _sc_scatter(g2, src, dst, zeros2)
    out = _tc3(aggp2, g2, dis, W2, b2[None, :])
    return out


# speedup vs baseline: 56.0841x; 1.0141x over previous
import functools

import jax
import jax.numpy as jnp
from jax import lax
from jax.experimental import pallas as pl
from jax.experimental.pallas import tpu as pltpu
from jax.experimental.pallas import tpu_sc as plsc

N_NODES = 10000
D_IN = 128
D_HID = 32
N_EDGES = 320000

NC = 2
NS = 16
NW = NC * NS
NPAD = 10240
ROWS_PER_TILE = NPAD // NS
CHUNK = 128
CH_PER_W = 80
EPAD = NW * CH_PER_W * CHUNK
RING = 10
DIST = 5
WROWS = NPAD * D_HID // 128

_sc_mesh = plsc.VectorSubcoreMesh(core_axis_name="c", subcore_axis_name="s")
_sc_params = pltpu.CompilerParams(use_tc_tiling_on_sc=False)


@functools.partial(
    pl.kernel,
    out_type=jax.ShapeDtypeStruct((NC, NPAD), jnp.float32),
    mesh=_sc_mesh,
    compiler_params=_sc_params,
    scratch_types=[
        pltpu.VMEM((CHUNK,), jnp.float32),
        pltpu.VMEM((CH_PER_W, CHUNK), jnp.int32),
        pltpu.VMEM_SHARED((NPAD,), jnp.float32),
    ],
)
def _sc_degree(dst2d_hbm, ones_hbm, zeros1_hbm, out_hbm, ones_v, dst_all, deg_sh):
    c = lax.axis_index("c")
    s = lax.axis_index("s")
    w = c * NS + s
    pltpu.sync_copy(zeros1_hbm, deg_sh.at[pl.ds(s * ROWS_PER_TILE, ROWS_PER_TILE)])
    pltpu.sync_copy(ones_hbm, ones_v)
    pltpu.sync_copy(dst2d_hbm.at[pl.ds(w * CH_PER_W, CH_PER_W)], dst_all)
    plsc.subcore_barrier()

    def body(it, carry):
        pltpu.sync_copy(ones_v, deg_sh.at[dst_all.at[it]], add=True)
        return carry

    lax.fori_loop(0, CH_PER_W, body, 0)
    plsc.subcore_barrier()
    sl = pl.ds(s * ROWS_PER_TILE, ROWS_PER_TILE)
    pltpu.sync_copy(deg_sh.at[sl], out_hbm.at[c, sl])


@functools.partial(
    pl.kernel,
    out_type=jax.ShapeDtypeStruct((NC, NPAD, D_HID), jnp.float32),
    mesh=_sc_mesh,
    compiler_params=_sc_params,
    scratch_types=[
        pltpu.VMEM((CH_PER_W, CHUNK), jnp.int32),
        pltpu.VMEM((CH_PER_W, CHUNK), jnp.int32),
        pltpu.VMEM((RING, CHUNK, D_HID), jnp.float32),
        pltpu.VMEM_SHARED((NPAD, D_HID), jnp.float32),
        pltpu.SemaphoreType.DMA((RING,)),
        pltpu.SemaphoreType.DMA((RING,)),
    ],
)
def _sc_scatter(g_hbm, src2d_hbm, dst2d_hbm, zeros2_hbm, out_hbm,
                src_all, dst_all, rows, agg_sh, gsem, ssem):
    c = lax.axis_index("c")
    s = lax.axis_index("s")
    w = c * NS + s
    pltpu.sync_copy(zeros2_hbm, agg_sh.at[pl.ds(s * ROWS_PER_TILE, ROWS_PER_TILE)])
    pltpu.sync_copy(src2d_hbm.at[pl.ds(w * CH_PER_W, CH_PER_W)], src_all)
    pltpu.sync_copy(dst2d_hbm.at[pl.ds(w * CH_PER_W, CH_PER_W)], dst_all)
    plsc.subcore_barrier()

    def gather_start(b, it):
        pltpu.async_copy(g_hbm.at[src_all.at[it]], rows.at[b], gsem.at[b])

    def gather_wait(b, it):
        pltpu.make_async_copy(g_hbm.at[src_all.at[it]], rows.at[b],
                              gsem.at[b]).wait()

    def scat_start(b, it):
        pltpu.async_copy(rows.at[b], agg_sh.at[dst_all.at[it]], ssem.at[b],
                         add=True)

    def scat_wait(b, it):
        pltpu.make_async_copy(rows.at[b], agg_sh.at[dst_all.at[it]],
                              ssem.at[b]).wait()

    for b in range(DIST):
        gather_start(b, b)

    def group(k, carry):
        for b in range(RING):
            it = k * RING + b
            gather_wait(b, it)
            scat_start(b, it)
            wb = (b + DIST) % RING
            if b >= DIST:
                scat_wait(wb, it - DIST)

                @pl.when(k < (CH_PER_W // RING) - 1)
                def _():
                    gather_start(wb, it + DIST)
            else:
                @pl.when(k > 0)
                def _():
                    scat_wait(wb, it - DIST)

                gather_start(wb, it + DIST)
        return carry

    lax.fori_loop(0, CH_PER_W // RING, group, 0)
    for it in range(CH_PER_W - DIST, CH_PER_W):
        scat_wait(it % RING, it)
    plsc.subcore_barrier()
    sl = pl.ds(s * ROWS_PER_TILE, ROWS_PER_TILE)
    pltpu.sync_copy(agg_sh.at[sl], out_hbm.at[c, sl])


def _tc1_body(x_ref, w1_ref, degpt_ref, g1_ref, dis_ref):
    deg = degpt_ref[:, 0:1] + degpt_ref[:, 1:2] + 1.0
    dis = lax.rsqrt(deg)
    dis_ref[...] = dis
    h = jnp.dot(x_ref[...], w1_ref[...], preferred_element_type=jnp.float32)
    g1_ref[0:N_NODES] = dis[:N_NODES] * h
    g1_ref[N_NODES:] = jnp.zeros((NPAD - N_NODES, D_HID), jnp.float32)


def _tc2_body(aggp_ref, g1_ref, dis_ref, b1_ref, g2_ref):
    dis = dis_ref[...]
    t = aggp_ref[0] + aggp_ref[1] + g1_ref[...]
    g2_ref[...] = dis * (dis * t + b1_ref[...])


def _tc3_body(aggp_ref, g2_ref, dis_ref, w2_ref, b2_ref, out_ref):
    t = aggp_ref[0] + aggp_ref[1] + g2_ref[...]
    pre = (dis_ref[...] * t)[:N_NODES]
    y = jnp.dot(pre, w2_ref[...], preferred_element_type=jnp.float32) + b2_ref[...]
    out_ref[...] = jnp.maximum(y, 0.0)


_tc1 = pl.pallas_call(
    _tc1_body,
    out_shape=(jax.ShapeDtypeStruct((NPAD, D_HID), jnp.float32),
               jax.ShapeDtypeStruct((NPAD, 1), jnp.float32)),
)
_tc2 = pl.pallas_call(
    _tc2_body,
    out_shape=jax.ShapeDtypeStruct((NPAD, D_HID), jnp.float32),
)
_tc3 = pl.pallas_call(
    _tc3_body,
    out_shape=jax.ShapeDtypeStruct((N_NODES, D_IN), jnp.float32),
)


def kernel(x, adj, W1, b1, W2, b2):
    x = x.astype(jnp.float32)
    pad = N_NODES + jnp.arange(EPAD - N_EDGES, dtype=jnp.int32) % (NPAD - N_NODES)
    src = jnp.concatenate([adj[0].astype(jnp.int32), pad]).reshape(
        NW * CH_PER_W, CHUNK)
    dst = jnp.concatenate([adj[1].astype(jnp.int32), pad]).reshape(
        NW * CH_PER_W, CHUNK)

    ones_ch = jnp.ones((CHUNK,), jnp.float32)
    zeros1 = jnp.zeros((ROWS_PER_TILE,), jnp.float32)
    zeros2 = jnp.zeros((ROWS_PER_TILE, D_HID), jnp.float32)

    degp = _sc_degree(dst, ones_ch, zeros1)
    degpt = degp.T

    g1, dis = _tc1(x, W1, degpt)
    aggp1 = _sc_scatter(g1, src, dst, zeros2)
    g2 = _tc2(aggp1, g1, dis, b1[None, :])
    aggp2 = _sc_scatter(g2, src, dst, zeros2)
    out = _tc3(aggp2, g2, dis, W2, b2[None, :])
    return out
